# Initial kernel scaffold; baseline (speedup 1.0000x reference)
import jax
import jax.numpy as jnp
from jax.experimental import pallas as pl


def kernel(boundary_logits, gtmasks, weight1, weight2):
    raise NotImplementedError("write your pallas kernel here")


# speedup vs baseline: 1.0370x; 1.0370x over previous
import functools

import jax
import jax.numpy as jnp
from jax.experimental import pallas as pl
from jax.experimental.pallas import tpu as pltpu


def _stats_kernel(x_ref, g_ref, t2_ref, tc_ref, stats_ref, *, D, P, L):
    x = x_ref[0]
    g = g_ref[0]
    zero = jnp.float32(0.0)

    g2 = g.reshape(D * P, L).astype(jnp.bfloat16)
    a = jax.lax.dot_general(g2, t2_ref[...], (((1,), (0,)), ((), ())),
                            preferred_element_type=jnp.float32).reshape(D, P, L)
    b = jax.lax.dot_general(g2, tc_ref[...], (((1,), (0,)), ((), ())),
                            preferred_element_type=jnp.float32).reshape(D, P, L)

    dd = jax.lax.broadcasted_iota(jnp.int32, (D, P, L), 0)
    pp = jax.lax.broadcasted_iota(jnp.int32, (D, P, L), 1)
    ll = jax.lax.broadcasted_iota(jnp.int32, (D, P, L), 2)
    half1 = ll >= (L // 2)
    p_first = pp == 0
    p_last = pp == (P - 1)
    d_first = dd == 0
    d_last = dd == (D - 1)

    up = pltpu.roll(b, shift=1, axis=1)
    down = pltpu.roll(b, shift=P - 1, axis=1)
    cross = jnp.where(half1,
                      jnp.where(p_last, zero, down),
                      jnp.where(p_first, zero, up))
    s2 = a + cross

    plus_d = jnp.where(d_last, zero, pltpu.roll(s2, shift=D - 1, axis=0))
    minus_d = jnp.where(d_first, zero, pltpu.roll(s2, shift=1, axis=0))
    box = s2 + plus_d + minus_d

    t = (27.0 * g - box > 0.1).astype(jnp.float32)

    xt = x * t
    inter = jnp.sum(xt)
    sum_x = jnp.sum(x)
    sum_t = jnp.sum(t)
    bp = jnp.sum(jnp.maximum(x, zero) + jnp.log(1.0 + jnp.exp(-jnp.abs(x))))

    lane = jax.lax.broadcasted_iota(jnp.int32, (1, 1, 128), 2)
    stats_ref[...] = (jnp.where(lane == 0, inter, zero)
                      + jnp.where(lane == 1, sum_x, zero)
                      + jnp.where(lane == 2, sum_t, zero)
                      + jnp.where(lane == 3, bp, zero))


def kernel(boundary_logits, gtmasks, weight1, weight2):
    N, C, D, H, W = boundary_logits.shape
    assert C == 1 and H % 2 == 0
    P = H // 2
    L = 2 * W

    x = boundary_logits.reshape(N, D, P, L).astype(jnp.float32)
    g = gtmasks.reshape(N, D, P, L).astype(jnp.float32)

    lv = jnp.arange(L)
    wv = lv % W
    hv = lv // W
    near = jnp.abs(wv[:, None] - wv[None, :]) <= 1
    t2 = near.astype(jnp.bfloat16)
    tc = (near & (hv[:, None] != hv[None, :])).astype(jnp.bfloat16)

    body = functools.partial(_stats_kernel, D=D, P=P, L=L)
    stats = pl.pallas_call(
        body,
        out_shape=jax.ShapeDtypeStruct((N, 1, 128), jnp.float32),
        grid_spec=pltpu.PrefetchScalarGridSpec(
            num_scalar_prefetch=0,
            grid=(N,),
            in_specs=[
                pl.BlockSpec((1, D, P, L), lambda n: (n, 0, 0, 0)),
                pl.BlockSpec((1, D, P, L), lambda n: (n, 0, 0, 0)),
                pl.BlockSpec((L, L), lambda n: (0, 0)),
                pl.BlockSpec((L, L), lambda n: (0, 0)),
            ],
            out_specs=pl.BlockSpec((1, 1, 128), lambda n: (n, 0, 0)),
        ),
        compiler_params=pltpu.CompilerParams(
            dimension_semantics=("parallel",),
            vmem_limit_bytes=32 * 1024 * 1024,
        ),
    )(x, g, t2, tc)

    inter = stats[:, 0, 0]
    sum_x = stats[:, 0, 1]
    sum_t = stats[:, 0, 2]
    bce_sum = stats[:, 0, 3] - inter

    eps = 1.0
    dice_coeff = jnp.mean(2.0 * inter / (sum_x + sum_t + eps))
    dice_loss = 1.0 - dice_coeff
    bce_loss = jnp.sum(bce_sum) / float(N * D * H * W)

    w1 = jnp.asarray(weight1, jnp.float32)
    w2 = jnp.asarray(weight2, jnp.float32)
    return (w1 ** -2) * bce_loss + (w2 ** -2) * dice_loss + jnp.log(1.0 + w1 * w2)


# speedup vs baseline: 1.3583x; 1.3099x over previous
import functools

import jax
import jax.numpy as jnp
from jax.experimental import pallas as pl
from jax.experimental.pallas import tpu as pltpu

_BN = 8


def _stats_kernel(x_ref, g_ref, t2_ref, tc0_ref, tc1_ref, stats_ref, *,
                  BN, D, P, L):
    x = x_ref[...]
    g = g_ref[...]
    zero = jnp.float32(0.0)

    g2 = g.reshape(BN * D * P, L).astype(jnp.bfloat16)
    dims = (((1,), (0,)), ((), ()))
    a = jax.lax.dot_general(g2, t2_ref[...], dims,
                            preferred_element_type=jnp.float32)
    b0 = jax.lax.dot_general(g2, tc0_ref[...], dims,
                             preferred_element_type=jnp.float32)
    b1 = jax.lax.dot_general(g2, tc1_ref[...], dims,
                             preferred_element_type=jnp.float32)
    a = a.reshape(BN, D, P, L)
    b0 = b0.reshape(BN, D, P, L)
    b1 = b1.reshape(BN, D, P, L)

    dd = jax.lax.broadcasted_iota(jnp.int32, (BN, D, P, L), 1)
    pp = jax.lax.broadcasted_iota(jnp.int32, (BN, D, P, L), 2)
    p_first = pp == 0
    p_last = pp == (P - 1)
    d_first = dd == 0
    d_last = dd == (D - 1)

    up = pltpu.roll(b0, shift=1, axis=2)
    down = pltpu.roll(b1, shift=P - 1, axis=2)
    s2 = a + jnp.where(p_first, zero, up) + jnp.where(p_last, zero, down)

    plus_d = jnp.where(d_last, zero, pltpu.roll(s2, shift=D - 1, axis=1))
    minus_d = jnp.where(d_first, zero, pltpu.roll(s2, shift=1, axis=1))
    box = s2 + plus_d + minus_d

    t = (27.0 * g - box > 0.1).astype(jnp.float32)

    xt = x * t
    bce_part = jnp.maximum(x, zero) + jnp.log(1.0 + jnp.exp(-jnp.abs(x)))

    lane = jax.lax.broadcasted_iota(jnp.int32, (1, 128), 1)
    for i in range(BN):
        inter = jnp.sum(xt[i])
        sum_x = jnp.sum(x[i])
        sum_t = jnp.sum(t[i])
        bp = jnp.sum(bce_part[i])
        stats_ref[i] = (jnp.where(lane == 0, inter, zero)
                        + jnp.where(lane == 1, sum_x, zero)
                        + jnp.where(lane == 2, sum_t, zero)
                        + jnp.where(lane == 3, bp, zero))


def kernel(boundary_logits, gtmasks, weight1, weight2):
    N, C, D, H, W = boundary_logits.shape
    assert C == 1 and H % 2 == 0
    P = H // 2
    L = 2 * W
    BN = _BN if N % _BN == 0 else 1

    x = boundary_logits.reshape(N, D, P, L).astype(jnp.float32)
    g = gtmasks.reshape(N, D, P, L).astype(jnp.float32)

    lv = jnp.arange(L)
    wv = lv % W
    hv = lv // W
    near = jnp.abs(wv[:, None] - wv[None, :]) <= 1
    diff_half = hv[:, None] != hv[None, :]
    t2 = near.astype(jnp.bfloat16)
    tc0 = (near & diff_half & (hv[None, :] == 0)).astype(jnp.bfloat16)
    tc1 = (near & diff_half & (hv[None, :] == 1)).astype(jnp.bfloat16)

    body = functools.partial(_stats_kernel, BN=BN, D=D, P=P, L=L)
    stats = pl.pallas_call(
        body,
        out_shape=jax.ShapeDtypeStruct((N, 1, 128), jnp.float32),
        grid_spec=pltpu.PrefetchScalarGridSpec(
            num_scalar_prefetch=0,
            grid=(N // BN,),
            in_specs=[
                pl.BlockSpec((BN, D, P, L), lambda n: (n, 0, 0, 0)),
                pl.BlockSpec((BN, D, P, L), lambda n: (n, 0, 0, 0)),
                pl.BlockSpec((L, L), lambda n: (0, 0)),
                pl.BlockSpec((L, L), lambda n: (0, 0)),
                pl.BlockSpec((L, L), lambda n: (0, 0)),
            ],
            out_specs=pl.BlockSpec((BN, 1, 128), lambda n: (n, 0, 0)),
        ),
        compiler_params=pltpu.CompilerParams(
            dimension_semantics=("parallel",),
            vmem_limit_bytes=56 * 1024 * 1024,
        ),
    )(x, g, t2, tc0, tc1)

    inter = stats[:, 0, 0]
    sum_x = stats[:, 0, 1]
    sum_t = stats[:, 0, 2]
    bce_sum = stats[:, 0, 3] - inter

    eps = 1.0
    dice_coeff = jnp.mean(2.0 * inter / (sum_x + sum_t + eps))
    dice_loss = 1.0 - dice_coeff
    bce_loss = jnp.sum(bce_sum) / float(N * D * H * W)

    w1 = jnp.asarray(weight1, jnp.float32)
    w2 = jnp.asarray(weight2, jnp.float32)
    return (w1 ** -2) * bce_loss + (w2 ** -2) * dice_loss + jnp.log(1.0 + w1 * w2)
